# initial kernel scaffold (unmeasured)
import jax
import jax.numpy as jnp
from jax import lax
from jax.experimental import pallas as pl
from jax.experimental.pallas import tpu as pltpu

N_DEV = 4
HQ = 8
DH = 128
SQ = 2048
D_MODEL = 1024
WINDOW = 128
SCALE = 0.08838834764831843
QBLK = 512
CHUNK = SQ // N_DEV
N_HOPS = 2 * (N_DEV - 1)


def kernel(x, Wq, K_ext, V_ext, Wo):
    r = lax.axis_index("i")
    K = lax.dynamic_slice_in_dim(K_ext[0], r * HQ, HQ, axis=1)
    V = lax.dynamic_slice_in_dim(V_ext[0], r * HQ, HQ, axis=1)

    def body(x_ref, wq_ref, k_ref, v_ref, wo_ref, out_ref,
             ctx_ref, acc_ref, comm_ref, sbuf_ref, send_sems, recv_sems):
        my = lax.axis_index("i")
        left = (my + N_DEV - 1) % N_DEV
        right = (my + 1) % N_DEV

        barrier = pltpu.get_barrier_semaphore()
        for nbr in (left, right):
            pl.semaphore_signal(barrier, inc=1, device_id=(nbr,),
                                device_id_type=pl.DeviceIdType.MESH)
        pl.semaphore_wait(barrier, 2)

        xb = x_ref[0].astype(jnp.bfloat16)
        wqb = wq_ref[...].astype(jnp.bfloat16)
        q = jnp.dot(xb, wqb, preferred_element_type=jnp.float32)
        q = (q * SCALE).astype(jnp.bfloat16)

        for h in range(HQ):
            kh = k_ref[:, h, :].astype(jnp.bfloat16)
            vh = v_ref[:, h, :].astype(jnp.bfloat16)
            for qb in range(SQ // QBLK):
                q0 = qb * QBLK
                k0 = max(0, q0 - WINDOW)
                k1 = min(SQ, q0 + QBLK + WINDOW)
                qblk = q[q0:q0 + QBLK, h * DH:(h + 1) * DH]
                s = lax.dot_general(
                    qblk, kh[k0:k1, :], (((1,), (1,)), ((), ())),
                    preferred_element_type=jnp.float32)
                ii = q0 + lax.broadcasted_iota(jnp.int32, s.shape, 0)
                jj = k0 + lax.broadcasted_iota(jnp.int32, s.shape, 1)
                s = jnp.where(jnp.abs(ii - jj) <= WINDOW, s, -1e9)
                mx = jnp.max(s, axis=1, keepdims=True)
                w = jnp.exp(s - mx)
                p = (w / jnp.sum(w, axis=1, keepdims=True)).astype(jnp.bfloat16)
                ctx = jnp.dot(p, vh[k0:k1, :],
                              preferred_element_type=jnp.float32)
                ctx_ref[q0:q0 + QBLK, h * DH:(h + 1) * DH] = (
                    ctx.astype(jnp.bfloat16))

        wob = wo_ref[...].astype(jnp.bfloat16)
        acc_ref[...] = jnp.dot(ctx_ref[...], wob,
                               preferred_element_type=jnp.float32)

        for s_ in range(N_DEV - 1):
            cs = (my - s_ + N_DEV) % N_DEV
            sbuf_ref[...] = acc_ref[pl.ds(cs * CHUNK, CHUNK), :].astype(
                jnp.bfloat16)
            rdma = pltpu.make_async_remote_copy(
                src_ref=sbuf_ref,
                dst_ref=comm_ref.at[s_],
                send_sem=send_sems.at[s_],
                recv_sem=recv_sems.at[s_],
                device_id=(right,),
                device_id_type=pl.DeviceIdType.MESH)
            rdma.start()
            rdma.wait()
            cr = (my - s_ - 1 + N_DEV) % N_DEV
            acc_ref[pl.ds(cr * CHUNK, CHUNK), :] = (
                acc_ref[pl.ds(cr * CHUNK, CHUNK), :]
                + comm_ref[s_].astype(jnp.float32))

        own = (my + 1) % N_DEV
        out_ref[0, pl.ds(own * CHUNK, CHUNK), :] = (
            acc_ref[pl.ds(own * CHUNK, CHUNK), :])

        for g in range(N_DEV - 1):
            hop = (N_DEV - 1) + g
            if g == 0:
                sbuf_ref[...] = acc_ref[pl.ds(own * CHUNK, CHUNK), :].astype(
                    jnp.bfloat16)
                src = sbuf_ref
            else:
                src = comm_ref.at[hop - 1]
            rdma = pltpu.make_async_remote_copy(
                src_ref=src,
                dst_ref=comm_ref.at[hop],
                send_sem=send_sems.at[hop],
                recv_sem=recv_sems.at[hop],
                device_id=(right,),
                device_id_type=pl.DeviceIdType.MESH)
            rdma.start()
            rdma.wait()
            cr = (my - g + N_DEV) % N_DEV
            out_ref[0, pl.ds(cr * CHUNK, CHUNK), :] = (
                comm_ref[hop].astype(jnp.float32))

    out_shape = jax.ShapeDtypeStruct((1, SQ, D_MODEL), jnp.float32)
    return pl.pallas_call(
        body,
        out_shape=out_shape,
        in_specs=[pl.BlockSpec(memory_space=pltpu.VMEM)] * 5,
        out_specs=pl.BlockSpec(memory_space=pltpu.VMEM),
        scratch_shapes=[
            pltpu.VMEM((SQ, HQ * DH), jnp.bfloat16),
            pltpu.VMEM((SQ, D_MODEL), jnp.float32),
            pltpu.VMEM((N_HOPS, CHUNK, D_MODEL), jnp.bfloat16),
            pltpu.VMEM((CHUNK, D_MODEL), jnp.bfloat16),
            pltpu.SemaphoreType.DMA((N_HOPS,)),
            pltpu.SemaphoreType.DMA((N_HOPS,)),
        ],
        compiler_params=pltpu.CompilerParams(collective_id=0),
    )(x, Wq, K, V, Wo)


# baseline (device time: 163770 ns/iter reference)
import jax
import jax.numpy as jnp
from jax import lax
from jax.experimental import pallas as pl
from jax.experimental.pallas import tpu as pltpu

N_DEV = 4
HQ = 8
DH = 128
SQ = 2048
D_MODEL = 1024
WINDOW = 128
SCALE = 0.08838834764831843
QBLK = 512
CHUNK = SQ // N_DEV
N_HOPS = 2 * (N_DEV - 1)


def kernel(x, Wq, K_ext, V_ext, Wo):
    r = lax.axis_index("i")
    K = lax.dynamic_slice_in_dim(K_ext[0], r * HQ, HQ, axis=1)
    V = lax.dynamic_slice_in_dim(V_ext[0], r * HQ, HQ, axis=1)
    xb = x[0].astype(jnp.bfloat16)
    wqb = Wq.astype(jnp.bfloat16)
    wob = Wo.astype(jnp.bfloat16)
    kb = K.astype(jnp.bfloat16)
    vb = V.astype(jnp.bfloat16)

    def body(x_ref, wq_ref, k_ref, v_ref, wo_ref, out_ref,
             ctx_ref, comm_ref, sbuf_ref, send_sems, recv_sems):
        my = lax.axis_index("i")
        left = (my + N_DEV - 1) % N_DEV
        right = (my + 1) % N_DEV

        barrier = pltpu.get_barrier_semaphore()
        for nbr in (left, right):
            pl.semaphore_signal(barrier, inc=1, device_id=(nbr,),
                                device_id_type=pl.DeviceIdType.MESH)
        pl.semaphore_wait(barrier, 2)

        q = jnp.dot(x_ref[...], wq_ref[...],
                    preferred_element_type=jnp.float32)
        q = (q * SCALE).astype(jnp.bfloat16)

        for h in range(HQ):
            kh = k_ref[:, h, :]
            vh = v_ref[:, h, :]
            for qb in range(SQ // QBLK):
                q0 = qb * QBLK
                k0 = max(0, q0 - WINDOW)
                k1 = min(SQ, q0 + QBLK + WINDOW)
                qblk = q[q0:q0 + QBLK, h * DH:(h + 1) * DH]
                s = lax.dot_general(
                    qblk, kh[k0:k1, :], (((1,), (1,)), ((), ())),
                    preferred_element_type=jnp.float32)
                ii = q0 + lax.broadcasted_iota(jnp.int32, s.shape, 0)
                jj = k0 + lax.broadcasted_iota(jnp.int32, s.shape, 1)
                s = jnp.where(jnp.abs(ii - jj) <= WINDOW, s, -1e9)
                mx = jnp.max(s, axis=1, keepdims=True)
                w = jnp.exp(s - mx)
                p = (w / jnp.sum(w, axis=1, keepdims=True)).astype(jnp.bfloat16)
                ctx = jnp.dot(p, vh[k0:k1, :],
                              preferred_element_type=jnp.float32)
                ctx_ref[q0:q0 + QBLK, h * DH:(h + 1) * DH] = (
                    ctx.astype(jnp.bfloat16))

        out_ref[0] = jnp.dot(ctx_ref[...], wo_ref[...],
                             preferred_element_type=jnp.float32)

        for s_ in range(N_DEV - 1):
            cs = (my - s_ + N_DEV) % N_DEV
            sbuf_ref[...] = out_ref[0, pl.ds(cs * CHUNK, CHUNK), :].astype(
                jnp.bfloat16)
            rdma = pltpu.make_async_remote_copy(
                src_ref=sbuf_ref,
                dst_ref=comm_ref.at[s_],
                send_sem=send_sems.at[s_],
                recv_sem=recv_sems.at[s_],
                device_id=(right,),
                device_id_type=pl.DeviceIdType.MESH)
            rdma.start()
            rdma.wait()
            cr = (my - s_ - 1 + N_DEV) % N_DEV
            out_ref[0, pl.ds(cr * CHUNK, CHUNK), :] = (
                out_ref[0, pl.ds(cr * CHUNK, CHUNK), :]
                + comm_ref[s_].astype(jnp.float32))

        own = (my + 1) % N_DEV
        for g in range(N_DEV - 1):
            hop = (N_DEV - 1) + g
            if g == 0:
                sbuf_ref[...] = out_ref[0, pl.ds(own * CHUNK, CHUNK),
                                        :].astype(jnp.bfloat16)
                src = sbuf_ref
            else:
                src = comm_ref.at[hop - 1]
            rdma = pltpu.make_async_remote_copy(
                src_ref=src,
                dst_ref=comm_ref.at[hop],
                send_sem=send_sems.at[hop],
                recv_sem=recv_sems.at[hop],
                device_id=(right,),
                device_id_type=pl.DeviceIdType.MESH)
            rdma.start()
            rdma.wait()
            cr = (my - g + N_DEV) % N_DEV
            out_ref[0, pl.ds(cr * CHUNK, CHUNK), :] = (
                comm_ref[hop].astype(jnp.float32))

    out_shape = jax.ShapeDtypeStruct((1, SQ, D_MODEL), jnp.float32)
    return pl.pallas_call(
        body,
        out_shape=out_shape,
        in_specs=[pl.BlockSpec(memory_space=pltpu.VMEM)] * 5,
        out_specs=pl.BlockSpec(memory_space=pltpu.VMEM),
        scratch_shapes=[
            pltpu.VMEM((SQ, HQ * DH), jnp.bfloat16),
            pltpu.VMEM((N_HOPS, CHUNK, D_MODEL), jnp.bfloat16),
            pltpu.VMEM((CHUNK, D_MODEL), jnp.bfloat16),
            pltpu.SemaphoreType.DMA((N_HOPS,)),
            pltpu.SemaphoreType.DMA((N_HOPS,)),
        ],
        compiler_params=pltpu.CompilerParams(
            collective_id=0, vmem_limit_bytes=56 * 1024 * 1024),
    )(xb, wqb, kb, vb, wob)


# device time: 130467 ns/iter; 1.2553x vs baseline; 1.2553x over previous
import jax
import jax.numpy as jnp
from jax import lax
from jax.experimental import pallas as pl
from jax.experimental.pallas import tpu as pltpu

N_DEV = 4
HQ = 8
DH = 128
SQ = 2048
D_MODEL = 1024
WINDOW = 128
SCALE = 0.08838834764831843
QBLK = 512
CHUNK = SQ // N_DEV
HALF = D_MODEL // 2
N_HOPS = 2 * (N_DEV - 1)


def kernel(x, Wq, K_ext, V_ext, Wo):
    r = lax.axis_index("i")
    K = lax.dynamic_slice_in_dim(K_ext[0], r * HQ, HQ, axis=1)
    V = lax.dynamic_slice_in_dim(V_ext[0], r * HQ, HQ, axis=1)
    xb = x[0].astype(jnp.bfloat16)
    wqb = Wq.astype(jnp.bfloat16)
    wob = Wo.astype(jnp.bfloat16)
    kb = K.astype(jnp.bfloat16)
    vb = V.astype(jnp.bfloat16)

    def body(x_ref, wq_ref, k_ref, v_ref, wo_ref, out_ref,
             ctx_ref, comm_ref, sbuf_ref, send_sems, recv_sems):
        my = lax.axis_index("i")
        left = (my + N_DEV - 1) % N_DEV
        right = (my + 1) % N_DEV

        barrier = pltpu.get_barrier_semaphore()
        for nbr in (left, right):
            pl.semaphore_signal(barrier, inc=1, device_id=(nbr,),
                                device_id_type=pl.DeviceIdType.MESH)
        pl.semaphore_wait(barrier, 2)

        q = jnp.dot(x_ref[...], wq_ref[...],
                    preferred_element_type=jnp.float32)
        q = (q * SCALE).astype(jnp.bfloat16)

        for h in range(HQ):
            kh = k_ref[:, h, :]
            vh = v_ref[:, h, :]
            for qb in range(SQ // QBLK):
                q0 = qb * QBLK
                k0 = max(0, q0 - WINDOW)
                k1 = min(SQ, q0 + QBLK + WINDOW)
                qblk = q[q0:q0 + QBLK, h * DH:(h + 1) * DH]
                s = lax.dot_general(
                    qblk, kh[k0:k1, :], (((1,), (1,)), ((), ())),
                    preferred_element_type=jnp.float32)
                ii = q0 + lax.broadcasted_iota(jnp.int32, s.shape, 0)
                jj = k0 + lax.broadcasted_iota(jnp.int32, s.shape, 1)
                s = jnp.where(jnp.abs(ii - jj) <= WINDOW, s, -1e9)
                mx = jnp.max(s, axis=1, keepdims=True)
                w = jnp.exp(s - mx)
                p = (w / jnp.sum(w, axis=1, keepdims=True)).astype(jnp.bfloat16)
                ctx = jnp.dot(p, vh[k0:k1, :],
                              preferred_element_type=jnp.float32)
                ctx_ref[q0:q0 + QBLK, h * DH:(h + 1) * DH] = (
                    ctx.astype(jnp.bfloat16))

        out_ref[0] = jnp.dot(ctx_ref[...], wo_ref[...],
                             preferred_element_type=jnp.float32)

        def ring_rdma(ring, hop, src):
            dev = right if ring == 0 else left
            return pltpu.make_async_remote_copy(
                src_ref=src,
                dst_ref=comm_ref.at[ring, hop],
                send_sem=send_sems.at[ring, hop],
                recv_sem=recv_sems.at[ring, hop],
                device_id=(dev,),
                device_id_type=pl.DeviceIdType.MESH)

        for s_ in range(N_DEV - 1):
            cs0 = (my - s_ + N_DEV) % N_DEV
            cs1 = (my + s_) % N_DEV
            sbuf_ref[0] = out_ref[0, pl.ds(cs0 * CHUNK, CHUNK),
                                  0:HALF].astype(jnp.bfloat16)
            sbuf_ref[1] = out_ref[0, pl.ds(cs1 * CHUNK, CHUNK),
                                  HALF:D_MODEL].astype(jnp.bfloat16)
            r0 = ring_rdma(0, s_, sbuf_ref.at[0])
            r1 = ring_rdma(1, s_, sbuf_ref.at[1])
            r0.start()
            r1.start()
            r0.wait()
            r1.wait()
            cr0 = (my - s_ - 1 + N_DEV) % N_DEV
            cr1 = (my + s_ + 1) % N_DEV
            out_ref[0, pl.ds(cr0 * CHUNK, CHUNK), 0:HALF] = (
                out_ref[0, pl.ds(cr0 * CHUNK, CHUNK), 0:HALF]
                + comm_ref[0, s_].astype(jnp.float32))
            out_ref[0, pl.ds(cr1 * CHUNK, CHUNK), HALF:D_MODEL] = (
                out_ref[0, pl.ds(cr1 * CHUNK, CHUNK), HALF:D_MODEL]
                + comm_ref[1, s_].astype(jnp.float32))

        own0 = (my + 1) % N_DEV
        own1 = (my + N_DEV - 1) % N_DEV
        for g in range(N_DEV - 1):
            hop = (N_DEV - 1) + g
            if g == 0:
                sbuf_ref[0] = out_ref[0, pl.ds(own0 * CHUNK, CHUNK),
                                      0:HALF].astype(jnp.bfloat16)
                sbuf_ref[1] = out_ref[0, pl.ds(own1 * CHUNK, CHUNK),
                                      HALF:D_MODEL].astype(jnp.bfloat16)
                src0, src1 = sbuf_ref.at[0], sbuf_ref.at[1]
            else:
                src0 = comm_ref.at[0, hop - 1]
                src1 = comm_ref.at[1, hop - 1]
            r0 = ring_rdma(0, hop, src0)
            r1 = ring_rdma(1, hop, src1)
            r0.start()
            r1.start()
            r0.wait()
            r1.wait()
            cr0 = (my - g + N_DEV) % N_DEV
            cr1 = (my + g) % N_DEV
            out_ref[0, pl.ds(cr0 * CHUNK, CHUNK), 0:HALF] = (
                comm_ref[0, hop].astype(jnp.float32))
            out_ref[0, pl.ds(cr1 * CHUNK, CHUNK), HALF:D_MODEL] = (
                comm_ref[1, hop].astype(jnp.float32))

    out_shape = jax.ShapeDtypeStruct((1, SQ, D_MODEL), jnp.float32)
    return pl.pallas_call(
        body,
        out_shape=out_shape,
        in_specs=[pl.BlockSpec(memory_space=pltpu.VMEM)] * 5,
        out_specs=pl.BlockSpec(memory_space=pltpu.VMEM),
        scratch_shapes=[
            pltpu.VMEM((SQ, HQ * DH), jnp.bfloat16),
            pltpu.VMEM((2, N_HOPS, CHUNK, HALF), jnp.bfloat16),
            pltpu.VMEM((2, CHUNK, HALF), jnp.bfloat16),
            pltpu.SemaphoreType.DMA((2, N_HOPS)),
            pltpu.SemaphoreType.DMA((2, N_HOPS)),
        ],
        compiler_params=pltpu.CompilerParams(
            collective_id=0, vmem_limit_bytes=56 * 1024 * 1024),
    )(xb, wqb, kb, vb, wob)
